# baseline (speedup 1.0000x reference)
import numpy as np

import jax
import jax.numpy as jnp
from jax.experimental import pallas as pl
from jax.experimental.pallas import tpu as pltpu

_IMAGE = 512.0
_NUM_CLASSES = 80
_INF = 1e16
_RR = ((-1.0, 64.0), (64.0, 128.0), (128.0, 256.0), (256.0, 512.0), (512.0, 1e16))
_HW = ((64, 64), (32, 32), (16, 16), (8, 8), (4, 4))

_P = 5632
_N = sum(h * w for h, w in _HW)
_NB = -(-_N // _P)
_NP = _NB * _P
_GRP = 4


def _build_pixmeta() -> np.ndarray:
    m = np.zeros((24, _NP), np.float32)
    c = 0
    for (h, w), (r0, r1) in zip(_HW, _RR):
        s = np.float32(_IMAGE / h)
        inv = np.float32(1.0 / s)
        n = h * w
        ys, xs = np.meshgrid(np.arange(h), np.arange(w), indexing="ij")
        px = xs.ravel().astype(np.float32) * s
        py = ys.ravel().astype(np.float32) * s
        m[0:8, c:c + n] = px
        m[8:16, c:c + n] = py
        m[16, c:c + n] = px * inv
        m[17, c:c + n] = py * inv
        m[18, c:c + n] = -px * inv
        m[19, c:c + n] = -py * inv
        m[20, c:c + n] = -inv
        m[21, c:c + n] = -inv
        m[22, c:c + n] = inv
        m[23, c:c + n] = inv
        c += n
    return m


def _build_rr42() -> np.ndarray:
    m = np.full((16, 128), _INF, np.float32)
    base = 42 * 128
    c = 0
    for (h, w), (r0, r1) in zip(_HW, _RR):
        n = h * w
        lo, hi = max(c, base), min(c + n, base + 128)
        if lo < hi:
            m[0:8, lo - base:hi - base] = np.float32(r0)
            m[8:16, lo - base:hi - base] = np.float32(r1)
        c += n
    return m


_RR42 = _build_rr42()


_PIXMETA = _build_pixmeta()


def _match_body(gts_ref, tab_ref, meta_ref, rr42_ref, out_ref, bxs_ref):
    g = gts_ref[0]
    G = g.shape[0]
    P = out_ref.shape[2]
    nch = G // 8
    ncols = -(-_N // 128)

    for c in range(nch):
        gc = g[c * 8:(c + 1) * 8, :]
        x0 = jnp.broadcast_to(gc[:, 0:1], (8, 128))
        y0 = jnp.broadcast_to(gc[:, 1:2], (8, 128))
        x1 = jnp.broadcast_to(gc[:, 2:3], (8, 128))
        y1 = jnp.broadcast_to(gc[:, 3:4], (8, 128))
        cls = jnp.broadcast_to(gc[:, 4:5], (8, 128))
        r = slice(c * 8, (c + 1) * 8)
        bxs_ref[0, r, :] = x0
        bxs_ref[1, r, :] = y0
        bxs_ref[2, r, :] = x1
        bxs_ref[3, r, :] = y1
        bxs_ref[4, r, :] = jnp.where(
            cls >= 0.0, (x1 - x0) * (y1 - y0), _INF)

    io8 = jax.lax.broadcasted_iota(jnp.int32, (8, 128), 0)
    tab = tab_ref[0]
    ta, tb = tab[0:8, :], tab[8:16, :]

    for k0 in range(0, ncols, _GRP):
        ks = list(range(k0, min(k0 + _GRP, ncols)))
        csl = {k: slice(k * 128, (k + 1) * 128) for k in ks}
        pxs = {k: meta_ref[0:8, csl[k]] for k in ks}
        pys = {k: meta_ref[8:16, csl[k]] for k in ks}
        rr0s, rr1s = {}, {}
        for k in ks:
            lvl, c0 = None, 0
            for li, (h, w) in enumerate(_HW):
                n = h * w
                if c0 <= k * 128 and (k + 1) * 128 <= c0 + n:
                    lvl = li
                c0 += n
            if lvl is None:
                rr0s[k] = rr42_ref[0:8, :]
                rr1s[k] = rr42_ref[8:16, :]
            else:
                rr0s[k] = None if lvl == 0 else _RR[lvl][0]
                rr1s[k] = _RR[lvl][1]
        amin = {k: jnp.full((8, 128), _INF, jnp.float32) for k in ks}
        cid = {k: jnp.zeros((8, 128), jnp.int32) for k in ks}
        for c in range(nch):
            r = slice(c * 8, (c + 1) * 8)
            x0 = bxs_ref[0, r, :]
            y0 = bxs_ref[1, r, :]
            x1 = bxs_ref[2, r, :]
            y1 = bxs_ref[3, r, :]
            ar = bxs_ref[4, r, :]
            for k in ks:
                l = pxs[k] - x0
                t = pys[k] - y0
                rt = x1 - pxs[k]
                b = y1 - pys[k]
                mn = jnp.minimum(jnp.minimum(l, t), jnp.minimum(rt, b))
                mx = jnp.maximum(jnp.maximum(l, t), jnp.maximum(rt, b))
                upd = (ar < amin[k]) & (mn > 0.0) & (mx <= rr1s[k])
                if rr0s[k] is not None:
                    upd = upd & (mx >= rr0s[k])
                amin[k] = jnp.where(upd, ar, amin[k])
                cid[k] = jnp.where(upd, c, cid[k])

        for k in ks:
            av = amin[k]
            iv = cid[k] * 8 + io8
            for lev in (4, 2, 1):
                a0, a1 = av[0:lev, :], av[lev:2 * lev, :]
                i0_, i1_ = iv[0:lev, :], iv[lev:2 * lev, :]
                lt = a1 < a0
                im = jnp.where(lt, i1_, i0_)
                im = jnp.where(a0 == a1, jnp.minimum(i0_, i1_), im)
                av = jnp.minimum(a0, a1)
                iv = im
            half = iv < 128
            i0_ = jnp.where(half, iv, iv - 128)
            iib = jnp.broadcast_to(i0_, (8, 128))
            ga = jnp.take_along_axis(ta, iib, axis=1)
            gb = jnp.take_along_axis(tb, iib, axis=1)
            sv = jnp.where(jnp.broadcast_to(half, (8, 128)), ga, gb)
            cs = csl[k]
            out_ref[0, 0:4, cs] = (meta_ref[16:20, cs]
                                   + sv[0:4, :] * meta_ref[20:24, cs])
            out_ref[0, 4:5, cs] = jnp.where(
                av == _INF, float(_NUM_CLASSES), sv[4:5, :])


def kernel(feat0, feat1, feat2, feat3, feat4, gts):
    B, G = gts.shape[0], gts.shape[1]
    gt5 = jnp.pad(jnp.transpose(gts, (0, 2, 1)),
                  ((0, 0), (0, 3), (0, 256 - G)))
    tab = jnp.concatenate([gt5[:, :, 0:128], gt5[:, :, 128:256]], axis=1)
    out = pl.pallas_call(
        _match_body,
        grid=(B, _NB),
        in_specs=[
            pl.BlockSpec((1, G, 5), lambda i, j: (i, 0, 0)),
            pl.BlockSpec((1, 16, 128), lambda i, j: (i, 0, 0)),
            pl.BlockSpec((24, _P), lambda i, j: (0, j)),
            pl.BlockSpec((16, 128), lambda i, j: (0, 0)),
        ],
        out_specs=pl.BlockSpec((1, 8, _P), lambda i, j: (i, 0, j)),
        out_shape=jax.ShapeDtypeStruct((B, 8, _NP), jnp.float32),
        scratch_shapes=[pltpu.VMEM((5, G, 128), jnp.float32)],
        compiler_params=pltpu.CompilerParams(
            dimension_semantics=("parallel", "parallel"),
        ),
    )(gts, tab, jnp.asarray(_PIXMETA), jnp.asarray(_RR42))

    lab_all = out[:, 4, :]
    bts, labs = [], []
    c = 0
    for h, w in _HW:
        n = h * w
        bts.append(jnp.transpose(out[:, 0:4, c:c + n], (0, 2, 1))
                   .reshape(B, h, w, 4))
        labs.append(lab_all[:, c:c + n].reshape(B, h, w))
        c += n
    return tuple(bts) + tuple(labs)


# speedup vs baseline: 1.0802x; 1.0802x over previous
import numpy as np

import jax
import jax.numpy as jnp
from jax.experimental import pallas as pl
from jax.experimental.pallas import tpu as pltpu

_IMAGE = 512.0
_NUM_CLASSES = 80
_INF = 1e16
_RR = ((-1.0, 64.0), (64.0, 128.0), (128.0, 256.0), (256.0, 512.0), (512.0, 1e16))
_HW = ((64, 64), (32, 32), (16, 16), (8, 8), (4, 4))

_P = 5632
_N = sum(h * w for h, w in _HW)
_NB = -(-_N // _P)
_NP = _NB * _P
_GRP = 4


def _build_pixmeta() -> np.ndarray:
    m = np.zeros((24, _NP), np.float32)
    c = 0
    for (h, w), (r0, r1) in zip(_HW, _RR):
        s = np.float32(_IMAGE / h)
        inv = np.float32(1.0 / s)
        n = h * w
        ys, xs = np.meshgrid(np.arange(h), np.arange(w), indexing="ij")
        px = xs.ravel().astype(np.float32) * s
        py = ys.ravel().astype(np.float32) * s
        m[0:8, c:c + n] = px
        m[8:16, c:c + n] = py
        m[16, c:c + n] = px * inv
        m[17, c:c + n] = py * inv
        m[18, c:c + n] = -px * inv
        m[19, c:c + n] = -py * inv
        m[20, c:c + n] = -inv
        m[21, c:c + n] = -inv
        m[22, c:c + n] = inv
        m[23, c:c + n] = inv
        c += n
    return m


def _build_rr42() -> np.ndarray:
    m = np.full((16, 128), _INF, np.float32)
    base = 42 * 128
    c = 0
    for (h, w), (r0, r1) in zip(_HW, _RR):
        n = h * w
        lo, hi = max(c, base), min(c + n, base + 128)
        if lo < hi:
            m[0:8, lo - base:hi - base] = np.float32(r0)
            m[8:16, lo - base:hi - base] = np.float32(r1)
        c += n
    return m


_RR42 = _build_rr42()


_PIXMETA = _build_pixmeta()


def _match_body(gts_ref, tab_ref, meta_ref, rr42_ref, out_ref, bxs_ref):
    g = gts_ref[0]
    G = g.shape[0]
    P = out_ref.shape[2]
    nch = G // 8
    ncols = -(-_N // 128)

    for c in range(nch):
        gc = g[c * 8:(c + 1) * 8, :]
        x0 = jnp.broadcast_to(gc[:, 0:1], (8, 128))
        y0 = jnp.broadcast_to(gc[:, 1:2], (8, 128))
        x1 = jnp.broadcast_to(gc[:, 2:3], (8, 128))
        y1 = jnp.broadcast_to(gc[:, 3:4], (8, 128))
        cls = jnp.broadcast_to(gc[:, 4:5], (8, 128))
        r = slice(c * 8, (c + 1) * 8)
        bxs_ref[0, r, :] = x0
        bxs_ref[1, r, :] = y0
        bxs_ref[2, r, :] = x1
        bxs_ref[3, r, :] = y1
        bxs_ref[4, r, :] = jnp.where(
            cls >= 0.0, (x1 - x0) * (y1 - y0), _INF)

    io8 = jax.lax.broadcasted_iota(jnp.int32, (8, 128), 0)
    tab = tab_ref[0]
    ta, tb = tab[0:8, :], tab[8:16, :]

    for k0 in range(0, ncols, _GRP):
        ks = list(range(k0, min(k0 + _GRP, ncols)))
        csl = {k: slice(k * 128, (k + 1) * 128) for k in ks}
        pxs = {k: meta_ref[0:8, csl[k]] for k in ks}
        pys = {k: meta_ref[8:16, csl[k]] for k in ks}
        rr0s, rr1s = {}, {}
        for k in ks:
            lvl, c0 = None, 0
            for li, (h, w) in enumerate(_HW):
                n = h * w
                if c0 <= k * 128 and (k + 1) * 128 <= c0 + n:
                    lvl = li
                c0 += n
            if lvl is None:
                rr0s[k] = rr42_ref[0:8, :]
                rr1s[k] = rr42_ref[8:16, :]
            else:
                rr0s[k] = None if lvl == 0 else _RR[lvl][0]
                rr1s[k] = _RR[lvl][1]
        amin = {k: jnp.full((8, 128), _INF, jnp.float32) for k in ks}
        cid = {k: jnp.zeros((8, 128), jnp.int32) for k in ks}
        for c in range(nch):
            r = slice(c * 8, (c + 1) * 8)
            x0 = bxs_ref[0, r, :]
            y0 = bxs_ref[1, r, :]
            x1 = bxs_ref[2, r, :]
            y1 = bxs_ref[3, r, :]
            ar = bxs_ref[4, r, :]
            for k in ks:
                l = pxs[k] - x0
                t = pys[k] - y0
                rt = x1 - pxs[k]
                b = y1 - pys[k]
                mn = jnp.minimum(jnp.minimum(l, t), jnp.minimum(rt, b))
                mx = jnp.maximum(jnp.maximum(l, t), jnp.maximum(rt, b))
                upd = (ar < amin[k]) & (mn > 0.0) & (mx <= rr1s[k])
                if rr0s[k] is not None:
                    upd = upd & (mx >= rr0s[k])
                amin[k] = jnp.where(upd, ar, amin[k])
                cid[k] = jnp.where(upd, c, cid[k])

        for k in ks:
            av = amin[k]
            iv = cid[k] * 8 + io8
            for lev in (4, 2, 1):
                a0, a1 = av[0:lev, :], av[lev:2 * lev, :]
                i0_, i1_ = iv[0:lev, :], iv[lev:2 * lev, :]
                lt = a1 < a0
                im = jnp.where(lt, i1_, i0_)
                im = jnp.where(a0 == a1, jnp.minimum(i0_, i1_), im)
                av = jnp.minimum(a0, a1)
                iv = im
            half = iv < 128
            i0_ = jnp.where(half, iv, iv - 128)
            iib = jnp.broadcast_to(i0_, (8, 128))
            ga = jnp.take_along_axis(ta, iib, axis=1)
            gb = jnp.take_along_axis(tb, iib, axis=1)
            sv = jnp.where(jnp.broadcast_to(half, (8, 128)), ga, gb)
            cs = csl[k]
            out_ref[0, 0:4, cs] = (meta_ref[16:20, cs]
                                   + sv[0:4, :] * meta_ref[20:24, cs])
            out_ref[0, 4:5, cs] = jnp.where(
                av == _INF, float(_NUM_CLASSES), sv[4:5, :])


def kernel(feat0, feat1, feat2, feat3, feat4, gts):
    B, G = gts.shape[0], gts.shape[1]
    gt5 = jnp.pad(jnp.transpose(gts, (0, 2, 1)),
                  ((0, 0), (0, 3), (0, 256 - G)))
    tab = jnp.concatenate([gt5[:, :, 0:128], gt5[:, :, 128:256]], axis=1)
    out = pl.pallas_call(
        _match_body,
        grid=(B, _NB),
        in_specs=[
            pl.BlockSpec((1, G, 5), lambda i, j: (i, 0, 0)),
            pl.BlockSpec((1, 16, 128), lambda i, j: (i, 0, 0)),
            pl.BlockSpec((24, _P), lambda i, j: (0, j)),
            pl.BlockSpec((16, 128), lambda i, j: (0, 0)),
        ],
        out_specs=pl.BlockSpec((1, 8, _P), lambda i, j: (i, 0, j)),
        out_shape=jax.ShapeDtypeStruct((B, 8, _NP), jnp.float32),
        scratch_shapes=[pltpu.VMEM((5, G, 128), jnp.float32)],
        compiler_params=pltpu.CompilerParams(
            dimension_semantics=("parallel", "parallel"),
        ),
    )(gts, tab, jnp.asarray(_PIXMETA), jnp.asarray(_RR42))

    bt_all = jnp.transpose(out[:, 0:4, :], (0, 2, 1))
    lab_all = out[:, 4, :]
    bts, labs = [], []
    c = 0
    for h, w in _HW:
        n = h * w
        bts.append(bt_all[:, c:c + n, :].reshape(B, h, w, 4))
        labs.append(lab_all[:, c:c + n].reshape(B, h, w))
        c += n
    return tuple(bts) + tuple(labs)
